# baseline (speedup 1.0000x reference)
import functools
import jax
import jax.numpy as jnp
from jax import lax
from jax.experimental import pallas as pl
from jax.experimental.pallas import tpu as pltpu, tpu_sc as plsc

N = 1024
E = 16384
EF = 65536
C = 128
NB = 32
H = 128
S = 32
B = 16
NL = 4
NA = 2
CUTOFF = 5.0

BE = 2048
BF = 2048

_bf16 = jnp.bfloat16
_f32 = jnp.float32
_HI = lax.Precision.HIGHEST


def _sigmoid(x):
    return 1.0 / (1.0 + jnp.exp(-x))


def _silu(x):
    return x * _sigmoid(x)


def _dot_ref(a, b):
    return jnp.dot(a.astype(_bf16), b.astype(_bf16), preferred_element_type=_f32)


def _onehot_T(idx2d, rows, cols, dtype):
    return (lax.broadcasted_iota(jnp.int32, (rows, cols), 0) == idx2d).astype(dtype)


def _gather(ohT, table, precision=None):
    return lax.dot_general(ohT, table, (((0,), (0,)), ((), ())),
                           preferred_element_type=_f32, precision=precision)


def _gather2(ohT, hi, lo):
    return _gather(ohT, hi) + _gather(ohT, lo)


def _scatter(ohT, vals):
    return jnp.dot(ohT, vals, preferred_element_type=_f32)


def _split(x):
    hi = x.astype(_bf16)
    return hi, (x - hi.astype(_f32)).astype(_bf16)


def _edge_geom(oht_s, oht_d, pos_pad):
    vec = (_gather(oht_d.astype(_f32), pos_pad, _HI)
           - _gather(oht_s.astype(_f32), pos_pad, _HI))
    x, y, z = vec[:, 0:1], vec[:, 1:2], vec[:, 2:3]
    d = jnp.sqrt(x * x + y * y + z * z + 1e-12)
    return vec, d


def _rbf_block(d):
    n = lax.broadcasted_iota(jnp.int32, (1, NB), 1).astype(_f32) + 1.0
    xc = d / CUTOFF
    rbf = jnp.sqrt(2.0 / CUTOFF) * jnp.sin(n * (jnp.pi * xc)) / d
    u = jnp.clip(xc, 0.0, 1.0)
    fc = 1.0 - 10.0 * u ** 3 + 15.0 * u ** 4 - 6.0 * u ** 5
    return rbf * fc


def _rsh16(vec, d):
    u = vec / d
    x, y, z = u[:, 0:1], u[:, 1:2], u[:, 2:3]
    s3 = jnp.sqrt(3.0)
    cols = [jnp.ones_like(x), x, y, z, s3 * x * y, s3 * y * z,
            0.5 * (3.0 * z * z - 1.0), s3 * x * z, 0.5 * s3 * (x * x - y * y)]
    blk = x.shape[0]
    out = jnp.zeros((blk, 16), _f32)
    for k, c in enumerate(cols):
        sel = (lax.broadcasted_iota(jnp.int32, (1, 16), 1) == k).astype(_f32)
        out = out + c * sel
    return out


def _geom_e_body(src_ref, dst_ref, pos_ref, wf_ref, bf_ref, w_ref, rsh_ref):
    oht_s = _onehot_T(src_ref[0], N, BE, _f32)
    oht_d = _onehot_T(dst_ref[0], N, BE, _f32)
    vec, d = _edge_geom(oht_s, oht_d, pos_ref[...])
    rbf = _rbf_block(d)
    w_ref[...] = _silu(_dot_ref(rbf, wf_ref[...]) + bf_ref[...])
    rsh_ref[...] = _rsh16(vec, d)


def _geom_e(src3, dst3, pos_pad, wf_flat, b2d):
    nblk = E // BE
    return pl.pallas_call(
        _geom_e_body,
        grid=(nblk,),
        in_specs=[
            pl.BlockSpec((1, 1, BE), lambda i: (i, 0, 0)),
            pl.BlockSpec((1, 1, BE), lambda i: (i, 0, 0)),
            pl.BlockSpec((N, 8), lambda i: (0, 0)),
            pl.BlockSpec((NB, NL * C), lambda i: (0, 0)),
            pl.BlockSpec((1, NL * C), lambda i: (0, 0)),
        ],
        out_specs=[
            pl.BlockSpec((BE, NL * C), lambda i: (i, 0)),
            pl.BlockSpec((BE, 16), lambda i: (i, 0)),
        ],
        out_shape=[
            jax.ShapeDtypeStruct((E, NL * C), _f32),
            jax.ShapeDtypeStruct((E, 16), _f32),
        ],
    )(src3, dst3, pos_pad, wf_flat, b2d)


def _geom_ef_body(src_ref, dst_ref, pos_ref, s0h_ref, wc_ref, wp_ref,
                  fr2_ref, frsh_ref, pg_ref):
    oht_s = _onehot_T(src_ref[0], N, BF, _bf16)
    oht_d = _onehot_T(dst_ref[0], N, BF, _bf16)
    vec, d = _edge_geom(oht_s, oht_d, pos_ref[...])
    rbf = _rbf_block(d)
    fr2_ref[...] = _dot_ref(rbf, wc_ref[...])
    frsh_ref[...] = _rsh16(vec, d)
    n0sum = _gather(oht_s + oht_d, s0h_ref[...])
    pg_ref[...] = _silu(_dot_ref(n0sum, wp_ref[...]))


def _geom_ef(fsrc3, fdst3, pos_pad, s0_hi, wc_cat, wp):
    nblk = EF // BF
    return pl.pallas_call(
        _geom_ef_body,
        grid=(nblk,),
        in_specs=[
            pl.BlockSpec((1, 1, BF), lambda i: (i, 0, 0)),
            pl.BlockSpec((1, 1, BF), lambda i: (i, 0, 0)),
            pl.BlockSpec((N, 8), lambda i: (0, 0)),
            pl.BlockSpec((N, C), lambda i: (0, 0)),
            pl.BlockSpec((NB, NA * C), lambda i: (0, 0)),
            pl.BlockSpec((C, 9 * S), lambda i: (0, 0)),
        ],
        out_specs=[
            pl.BlockSpec((BF, NA * C), lambda i: (i, 0)),
            pl.BlockSpec((BF, 16), lambda i: (i, 0)),
            pl.BlockSpec((BF, 9 * S), lambda i: (i, 0)),
        ],
        out_shape=[
            jax.ShapeDtypeStruct((EF, NA * C), _f32),
            jax.ShapeDtypeStruct((EF, 16), _f32),
            jax.ShapeDtypeStruct((EF, 9 * S), _f32),
        ],
    )(fsrc3, fdst3, pos_pad, s0_hi, wc_cat, wp)


def _embed_body(at_ref, tab_ref, s0_ref, hi_ref, lo_ref):
    idx2d = jnp.reshape(at_ref[...], (1, N))
    oht = _onehot_T(idx2d, 128, N, _f32)
    s0 = _gather(oht, tab_ref[...], _HI)
    s0_ref[...] = s0
    hi, lo = _split(s0)
    hi_ref[...] = hi
    lo_ref[...] = lo


def _embed(at_no, embed_pad):
    return pl.pallas_call(
        _embed_body,
        in_specs=[pl.BlockSpec((N,), lambda: (0,)),
                  pl.BlockSpec((128, C), lambda: (0, 0))],
        out_specs=[pl.BlockSpec((N, C), lambda: (0, 0)),
                   pl.BlockSpec((N, C), lambda: (0, 0)),
                   pl.BlockSpec((N, C), lambda: (0, 0))],
        out_shape=[jax.ShapeDtypeStruct((N, C), _f32),
                   jax.ShapeDtypeStruct((N, C), _bf16),
                   jax.ShapeDtypeStruct((N, C), _bf16)],
        grid=(),
    )(at_no, embed_pad)


def _layer_pre_body(use_gate, s_ref, v_ref, ws_ref, wg_ref, hs_ref, lo_ref):
    hs = _dot_ref(s_ref[...], ws_ref[...])
    if use_gate:
        vn2 = jnp.zeros((N, C), _f32)
        for k in range(9):
            vk = v_ref[:, k * C:(k + 1) * C]
            vn2 = vn2 + vk * vk
        vn = jnp.sqrt(vn2 + 1e-6)
        hs = hs * _sigmoid(_dot_ref(vn, wg_ref[...]))
    hi, lo = _split(hs)
    hs_ref[...] = hi
    lo_ref[...] = lo


def _layer_pre(s, vflat, w_self_i, w_gate_i, use_gate):
    return pl.pallas_call(
        functools.partial(_layer_pre_body, use_gate),
        in_specs=[pl.BlockSpec((N, C), lambda: (0, 0)),
                  pl.BlockSpec((N, 9 * C), lambda: (0, 0)),
                  pl.BlockSpec((C, C), lambda: (0, 0)),
                  pl.BlockSpec((C, C), lambda: (0, 0))],
        out_specs=[pl.BlockSpec((N, C), lambda: (0, 0)),
                   pl.BlockSpec((N, C), lambda: (0, 0))],
        out_shape=[jax.ShapeDtypeStruct((N, C), _bf16),
                   jax.ShapeDtypeStruct((N, C), _bf16)],
        grid=(),
    )(s, vflat, w_self_i, w_gate_i)


def _edge_mv_body(do_v, src_ref, w_ref, rsh_ref, hs_ref, hslo_ref, mv_ref):
    oht_s = _onehot_T(src_ref[0], N, BE, _bf16)
    hsg = _gather2(oht_s, hs_ref[...], hslo_ref[...])
    m = w_ref[...] * hsg
    if do_v:
        rsh = rsh_ref[...]
        mv_ref[...] = jnp.concatenate(
            [m * rsh[:, k:k + 1] for k in range(9)] + [m], axis=1)
    else:
        mv_ref[...] = m


def _edge_mv(src3, w_all, rsh_e, hs_bf, hs_lo, layer, do_v):
    nblk = E // BE
    width = 10 * C if do_v else C
    return pl.pallas_call(
        functools.partial(_edge_mv_body, do_v),
        grid=(nblk,),
        in_specs=[
            pl.BlockSpec((1, 1, BE), lambda i: (i, 0, 0)),
            pl.BlockSpec((BE, C), lambda i, L=layer: (i, L)),
            pl.BlockSpec((BE, 16), lambda i: (i, 0)),
            pl.BlockSpec((N, C), lambda i: (0, 0)),
            pl.BlockSpec((N, C), lambda i: (0, 0)),
        ],
        out_specs=pl.BlockSpec((BE, width), lambda i: (i, 0)),
        out_shape=jax.ShapeDtypeStruct((E, width), _f32),
    )(src3, w_all, rsh_e, hs_bf, hs_lo)


_SC_K = 128


def _sc_scatter(R, NR):
    info = plsc.get_sparse_core_info()
    NC, NS = info.num_cores, info.num_subcores
    NW = NC * NS
    assert R % (NW * _SC_K) == 0 and NR % NS == 0
    rows_tile = R // NW
    iters = rows_tile // _SC_K
    acc_rows = NR // NS
    mesh = plsc.VectorSubcoreMesh(core_axis_name="c", subcore_axis_name="s")

    @functools.partial(
        pl.kernel, mesh=mesh,
        out_type=jax.ShapeDtypeStruct((NC, NR, C), jnp.float32),
        scratch_types=[
            pltpu.VMEM((_SC_K,), jnp.int32),
            pltpu.VMEM((_SC_K, C), jnp.float32),
            pltpu.VMEM_SHARED((NR, C), jnp.float32),
        ],
    )
    def k(vals_hbm, idx_hbm, zeros_hbm, out_hbm, ibuf, vbuf, shared):
        cid = lax.axis_index("c")
        sid = lax.axis_index("s")
        wid = sid * NC + cid
        pltpu.sync_copy(zeros_hbm.at[pl.ds(sid * acc_rows, acc_rows)],
                        shared.at[pl.ds(sid * acc_rows, acc_rows)])
        plsc.subcore_barrier()

        def body(it, _):
            base = wid * rows_tile + it * _SC_K
            pltpu.sync_copy(idx_hbm.at[pl.ds(base, _SC_K)], ibuf)
            pltpu.sync_copy(vals_hbm.at[pl.ds(base, _SC_K)], vbuf)
            pltpu.sync_copy(vbuf, shared.at[ibuf], add=True)
            return _

        lax.fori_loop(0, iters, body, 0)
        plsc.subcore_barrier()
        pltpu.sync_copy(shared.at[pl.ds(sid * acc_rows, acc_rows)],
                        out_hbm.at[cid, pl.ds(sid * acc_rows, acc_rows)])

    return k


def _layer_post_body(tail, has_v, s_ref, v_ref, p0_ref, p1_ref, wu1_ref,
                     wu2_ref, wab_ref, wn1_ref, wn2_ref, *out_refs):
    refs = list(out_refs)
    s_out = refs.pop(0)
    if has_v:
        agg_s = p0_ref[:, 9 * C:] + p1_ref[:, 9 * C:]
        v_out = refs.pop(0)
        v_out[...] = v_ref[...] + (p0_ref[:, :9 * C] + p1_ref[:, :9 * C])
    else:
        agg_s = p0_ref[...] + p1_ref[...]
    up = _silu(_dot_ref(agg_s, wu1_ref[...]))
    s_new = s_ref[...] + _dot_ref(up, wu2_ref[...])
    s_out[...] = s_new
    if tail:
        a12h_ref, a12l_ref, hn_ref = refs
        a12 = _dot_ref(s_new, wab_ref[...])
        hi, lo = _split(a12)
        a12h_ref[...] = hi
        a12l_ref[...] = lo
        h1 = _silu(_dot_ref(s_new, wn1_ref[...]))
        hn_ref[...] = _dot_ref(h1, wn2_ref[...])


def _layer_post(s, vflat, p0, p1, wu1, wu2, wab, wn1, wn2, tail, has_v):
    W = 10 * C if has_v else C
    out_specs = [pl.BlockSpec((N, C), lambda: (0, 0))]
    out_shape = [jax.ShapeDtypeStruct((N, C), _f32)]
    if has_v:
        out_specs.append(pl.BlockSpec((N, 9 * C), lambda: (0, 0)))
        out_shape.append(jax.ShapeDtypeStruct((N, 9 * C), _f32))
    if tail:
        out_specs += [pl.BlockSpec((N, 2 * C), lambda: (0, 0)),
                      pl.BlockSpec((N, 2 * C), lambda: (0, 0)),
                      pl.BlockSpec((N, 9 * S), lambda: (0, 0))]
        out_shape += [jax.ShapeDtypeStruct((N, 2 * C), _bf16),
                      jax.ShapeDtypeStruct((N, 2 * C), _bf16),
                      jax.ShapeDtypeStruct((N, 9 * S), _f32)]
    return pl.pallas_call(
        functools.partial(_layer_post_body, tail, has_v),
        in_specs=[pl.BlockSpec((N, C), lambda: (0, 0)),
                  pl.BlockSpec((N, 9 * C), lambda: (0, 0)),
                  pl.BlockSpec((N, W), lambda: (0, 0)),
                  pl.BlockSpec((N, W), lambda: (0, 0)),
                  pl.BlockSpec((C, C), lambda: (0, 0)),
                  pl.BlockSpec((C, C), lambda: (0, 0)),
                  pl.BlockSpec((C, 2 * C), lambda: (0, 0)),
                  pl.BlockSpec((C, H), lambda: (0, 0)),
                  pl.BlockSpec((H, 9 * S), lambda: (0, 0))],
        out_specs=out_specs,
        out_shape=out_shape,
        grid=(),
    )(s, vflat, p0, p1, wu1, wu2, wab, wn1, wn2)


def _edge_he_body(src_ref, dst_ref, fr2_ref, frsh_ref, pg_ref,
                  ash_ref, adh_ref,
                  we2a_ref, we2b_ref, weo_ref, emat_ref, nacc_ref):
    i = pl.program_id(0)
    oht_s = _onehot_T(src_ref[0], N, BF, _bf16)
    oht_d = _onehot_T(dst_ref[0], N, BF, _bf16)
    gs = _gather(oht_s, ash_ref[...])
    gd = _gather(oht_d, adh_ref[...])
    fr2 = fr2_ref[...]
    g0 = gs[:, :C] + gd[:, :C] + fr2[:, :C]
    g1 = gs[:, C:] + gd[:, C:] + fr2[:, C:]
    he = _dot_ref(_silu(g0), we2a_ref[...]) + _dot_ref(_silu(g1), we2b_ref[...])
    frsh = frsh_ref[...]
    acc = jnp.concatenate(
        [he[:, k * S:(k + 1) * S] * frsh[:, k:k + 1] for k in range(9)], axis=1)
    emat_ref[...] = _dot_ref(acc * pg_ref[...], weo_ref[...])

    @pl.when(i == 0)
    def _():
        nacc_ref[...] = jnp.zeros_like(nacc_ref)

    nacc_ref[...] += _scatter(oht_d, acc.astype(_bf16))


def _edge_he(fsrc3, fdst3, fr2, frsh, pg, ash, adh, we2a, we2b, weo):
    nblk = EF // BF
    return pl.pallas_call(
        _edge_he_body,
        grid=(nblk,),
        in_specs=[
            pl.BlockSpec((1, 1, BF), lambda i: (i, 0, 0)),
            pl.BlockSpec((1, 1, BF), lambda i: (i, 0, 0)),
            pl.BlockSpec((BF, NA * C), lambda i: (i, 0)),
            pl.BlockSpec((BF, 16), lambda i: (i, 0)),
            pl.BlockSpec((BF, 9 * S), lambda i: (i, 0)),
            pl.BlockSpec((N, NA * C), lambda i: (0, 0)),
            pl.BlockSpec((N, NA * C), lambda i: (0, 0)),
            pl.BlockSpec((C, 9 * S), lambda i: (0, 0)),
            pl.BlockSpec((C, 9 * S), lambda i: (0, 0)),
            pl.BlockSpec((9 * S, B * B), lambda i: (0, 0)),
        ],
        out_specs=[
            pl.BlockSpec((BF, B * B), lambda i: (i, 0)),
            pl.BlockSpec((N, 9 * S), lambda i: (0, 0)),
        ],
        out_shape=[
            jax.ShapeDtypeStruct((EF, B * B), _f32),
            jax.ShapeDtypeStruct((N, 9 * S), _f32),
        ],
    )(fsrc3, fdst3, fr2, frsh, pg, ash, adh, we2a, we2b, weo)


def _node_out_body(s0_ref, hn0_ref, hn1_ref, nacc_ref, wg0_ref, wno_ref, out_ref):
    node_sph = hn0_ref[...] + hn1_ref[...] + nacc_ref[...]
    g0 = _silu(_dot_ref(s0_ref[...], wg0_ref[...]))
    out_ref[...] = _dot_ref(node_sph * g0, wno_ref[...])


def _node_out(s0, hn0, hn1, nacc, wg0, wno):
    return pl.pallas_call(
        _node_out_body,
        in_specs=[pl.BlockSpec((N, C), lambda: (0, 0)),
                  pl.BlockSpec((N, 9 * S), lambda: (0, 0)),
                  pl.BlockSpec((N, 9 * S), lambda: (0, 0)),
                  pl.BlockSpec((N, 9 * S), lambda: (0, 0)),
                  pl.BlockSpec((C, 9 * S), lambda: (0, 0)),
                  pl.BlockSpec((9 * S, B * B), lambda: (0, 0))],
        out_specs=pl.BlockSpec((N, B * B), lambda: (0, 0)),
        out_shape=jax.ShapeDtypeStruct((N, B * B), _f32),
        grid=(),
    )(s0, hn0, hn1, nacc, wg0, wno)


def kernel(at_no, pos, edge_index, fc_edge_index, embed_table, W_filt, b_filt,
           W_self, W_gate, W_up1, W_up2, Wn1, Wn2, We1, We2, Wg0, Wnode_out,
           Wp, Wedge_out):
    src3 = edge_index[0].reshape(E // BE, 1, BE).astype(jnp.int32)
    dst3 = edge_index[1].reshape(E // BE, 1, BE).astype(jnp.int32)
    fsrc3 = fc_edge_index[0].reshape(EF // BF, 1, BF).astype(jnp.int32)
    fdst3 = fc_edge_index[1].reshape(EF // BF, 1, BF).astype(jnp.int32)
    pos_pad = jnp.zeros((N, 8), _f32).at[:, :3].set(pos)
    embed_pad = jnp.zeros((128, C), _f32).at[:100].set(embed_table)
    wf_flat = jnp.transpose(W_filt, (1, 0, 2)).reshape(NB, NL * C)
    b2d = b_filt.reshape(1, NL * C)
    wc_cat = jnp.transpose(We1[:, 2 * C:, :], (1, 0, 2)).reshape(NB, NA * C)

    s0, s0_hi, s0_lo = _embed(at_no.astype(jnp.int32), embed_pad)

    w_all, rsh_e = _geom_e(src3, dst3, pos_pad, wf_flat, b2d)
    del s0_lo
    fr2, frsh, pg = _geom_ef(fsrc3, fdst3, pos_pad, s0_hi, wc_cat, Wp)

    dst_i32 = edge_index[1].astype(jnp.int32)
    idx10 = (dst_i32[:, None] * 10
             + jnp.arange(10, dtype=jnp.int32)[None, :]).reshape(E * 10)
    zeros10 = jnp.zeros((10 * N, C), _f32)
    zeros1 = jnp.zeros((N, C), _f32)
    scat10 = _sc_scatter(E * 10, 10 * N)
    scat1 = _sc_scatter(E, N)

    s = s0
    vflat = jnp.zeros((N, 9 * C), _f32)
    a12h, a12l, hn = [], [], []
    for idx in range(NL):
        has_v = idx < NL - 1
        hs_bf, hs_lo = _layer_pre(s, vflat, W_self[idx], W_gate[idx],
                                  use_gate=idx > 0)
        mv = _edge_mv(src3, w_all, rsh_e, hs_bf, hs_lo, idx, do_v=has_v)
        if has_v:
            parts = scat10(mv.reshape(E * 10, C), idx10, zeros10)
            p0 = parts[0].reshape(N, 10 * C)
            p1 = parts[1].reshape(N, 10 * C)
        else:
            parts = scat1(mv, dst_i32, zeros1)
            p0, p1 = parts[0], parts[1]
        tail = idx >= NL - NA
        j = idx - (NL - NA)
        wab = (jnp.concatenate([We1[j, :C, :], We1[j, C:2 * C, :]], axis=1)
               if tail else jnp.zeros((C, 2 * C), _f32))
        outs = _layer_post(
            s, vflat, p0, p1, W_up1[idx], W_up2[idx], wab,
            Wn1[j] if tail else jnp.zeros((C, H), _f32),
            Wn2[j] if tail else jnp.zeros((H, 9 * S), _f32), tail, has_v)
        outs = list(outs)
        s = outs.pop(0)
        if has_v:
            vflat = outs.pop(0)
        if tail:
            a12h_i, a12l_i, hn_i = outs
            a12h.append(a12h_i)
            a12l.append(a12l_i)
            hn.append(hn_i)

    del a12l
    ash = jnp.concatenate([a12h[0][:, :C], a12h[1][:, :C]], axis=1)
    adh = jnp.concatenate([a12h[0][:, C:], a12h[1][:, C:]], axis=1)
    emat, nacc = _edge_he(fsrc3, fdst3, fr2, frsh, pg, ash, adh,
                          We2[0], We2[1], Wedge_out)
    nmat = _node_out(s0, hn[0], hn[1], nacc, Wg0, Wnode_out)
    return nmat.reshape(N, B, B), emat.reshape(EF, B, B)


# speedup vs baseline: 1.0963x; 1.0963x over previous
import functools
import jax
import jax.numpy as jnp
from jax import lax
from jax.experimental import pallas as pl
from jax.experimental.pallas import tpu as pltpu, tpu_sc as plsc

N = 1024
E = 16384
EF = 65536
C = 128
NB = 32
H = 128
S = 32
B = 16
NL = 4
NA = 2
CUTOFF = 5.0

BE = 2048
BF = 2048

_bf16 = jnp.bfloat16
_f32 = jnp.float32
_HI = lax.Precision.HIGHEST


def _sigmoid(x):
    return 1.0 / (1.0 + jnp.exp(-x))


def _silu(x):
    return x * _sigmoid(x)


def _dot_ref(a, b):
    return jnp.dot(a.astype(_bf16), b.astype(_bf16), preferred_element_type=_f32)


def _onehot_T(idx2d, rows, cols, dtype):
    return (lax.broadcasted_iota(jnp.int32, (rows, cols), 0) == idx2d).astype(dtype)


def _gather(ohT, table, precision=None):
    return lax.dot_general(ohT, table, (((0,), (0,)), ((), ())),
                           preferred_element_type=_f32, precision=precision)


def _gather2(ohT, hi, lo):
    return _gather(ohT, hi) + _gather(ohT, lo)


def _scatter(ohT, vals):
    return jnp.dot(ohT, vals, preferred_element_type=_f32)


def _split(x):
    hi = x.astype(_bf16)
    return hi, (x - hi.astype(_f32)).astype(_bf16)


def _edge_geom(oht_s, oht_d, pos_pad):
    vec = (_gather(oht_d.astype(_f32), pos_pad, _HI)
           - _gather(oht_s.astype(_f32), pos_pad, _HI))
    x, y, z = vec[:, 0:1], vec[:, 1:2], vec[:, 2:3]
    d = jnp.sqrt(x * x + y * y + z * z + 1e-12)
    return vec, d


def _rbf_block(d):
    n = lax.broadcasted_iota(jnp.int32, (1, NB), 1).astype(_f32) + 1.0
    xc = d / CUTOFF
    rbf = jnp.sqrt(2.0 / CUTOFF) * jnp.sin(n * (jnp.pi * xc)) / d
    u = jnp.clip(xc, 0.0, 1.0)
    fc = 1.0 - 10.0 * u ** 3 + 15.0 * u ** 4 - 6.0 * u ** 5
    return rbf * fc


def _rsh16(vec, d):
    u = vec / d
    x, y, z = u[:, 0:1], u[:, 1:2], u[:, 2:3]
    s3 = jnp.sqrt(3.0)
    cols = [jnp.ones_like(x), x, y, z, s3 * x * y, s3 * y * z,
            0.5 * (3.0 * z * z - 1.0), s3 * x * z, 0.5 * s3 * (x * x - y * y)]
    blk = x.shape[0]
    out = jnp.zeros((blk, 16), _f32)
    for k, c in enumerate(cols):
        sel = (lax.broadcasted_iota(jnp.int32, (1, 16), 1) == k).astype(_f32)
        out = out + c * sel
    return out


def _geom_e_body(src_ref, dst_ref, pos_ref, wf_ref, bf_ref, w_ref, rsh_ref):
    oht_s = _onehot_T(src_ref[0], N, BE, _f32)
    oht_d = _onehot_T(dst_ref[0], N, BE, _f32)
    vec, d = _edge_geom(oht_s, oht_d, pos_ref[...])
    rbf = _rbf_block(d)
    w_ref[...] = _silu(_dot_ref(rbf, wf_ref[...]) + bf_ref[...])
    rsh_ref[...] = _rsh16(vec, d)


def _geom_e(src3, dst3, pos_pad, wf_flat, b2d):
    nblk = E // BE
    return pl.pallas_call(
        _geom_e_body,
        grid=(nblk,),
        in_specs=[
            pl.BlockSpec((1, 1, BE), lambda i: (i, 0, 0)),
            pl.BlockSpec((1, 1, BE), lambda i: (i, 0, 0)),
            pl.BlockSpec((N, 8), lambda i: (0, 0)),
            pl.BlockSpec((NB, NL * C), lambda i: (0, 0)),
            pl.BlockSpec((1, NL * C), lambda i: (0, 0)),
        ],
        out_specs=[
            pl.BlockSpec((BE, NL * C), lambda i: (i, 0)),
            pl.BlockSpec((BE, 16), lambda i: (i, 0)),
        ],
        out_shape=[
            jax.ShapeDtypeStruct((E, NL * C), _f32),
            jax.ShapeDtypeStruct((E, 16), _f32),
        ],
    )(src3, dst3, pos_pad, wf_flat, b2d)


def _geom_ef_body(src_ref, dst_ref, pos_ref, s0h_ref, wc_ref, wp_ref,
                  fr2_ref, frsh_ref, pg_ref):
    oht_s = _onehot_T(src_ref[0], N, BF, _bf16)
    oht_d = _onehot_T(dst_ref[0], N, BF, _bf16)
    vec, d = _edge_geom(oht_s, oht_d, pos_ref[...])
    rbf = _rbf_block(d)
    fr2_ref[...] = _dot_ref(rbf, wc_ref[...])
    frsh_ref[...] = _rsh16(vec, d)
    n0sum = _gather(oht_s + oht_d, s0h_ref[...])
    pg_ref[...] = _silu(_dot_ref(n0sum, wp_ref[...]))


def _geom_ef(fsrc3, fdst3, pos_pad, s0_hi, wc_cat, wp):
    nblk = EF // BF
    return pl.pallas_call(
        _geom_ef_body,
        grid=(nblk,),
        in_specs=[
            pl.BlockSpec((1, 1, BF), lambda i: (i, 0, 0)),
            pl.BlockSpec((1, 1, BF), lambda i: (i, 0, 0)),
            pl.BlockSpec((N, 8), lambda i: (0, 0)),
            pl.BlockSpec((N, C), lambda i: (0, 0)),
            pl.BlockSpec((NB, NA * C), lambda i: (0, 0)),
            pl.BlockSpec((C, 9 * S), lambda i: (0, 0)),
        ],
        out_specs=[
            pl.BlockSpec((BF, NA * C), lambda i: (i, 0)),
            pl.BlockSpec((BF, 16), lambda i: (i, 0)),
            pl.BlockSpec((BF, 9 * S), lambda i: (i, 0)),
        ],
        out_shape=[
            jax.ShapeDtypeStruct((EF, NA * C), _f32),
            jax.ShapeDtypeStruct((EF, 16), _f32),
            jax.ShapeDtypeStruct((EF, 9 * S), _f32),
        ],
    )(fsrc3, fdst3, pos_pad, s0_hi, wc_cat, wp)


def _embed_body(at_ref, tab_ref, s0_ref, hi_ref, lo_ref):
    idx2d = jnp.reshape(at_ref[...], (1, N))
    oht = _onehot_T(idx2d, 128, N, _f32)
    s0 = _gather(oht, tab_ref[...], _HI)
    s0_ref[...] = s0
    hi, lo = _split(s0)
    hi_ref[...] = hi
    lo_ref[...] = lo


def _embed(at_no, embed_pad):
    return pl.pallas_call(
        _embed_body,
        in_specs=[pl.BlockSpec((N,), lambda: (0,)),
                  pl.BlockSpec((128, C), lambda: (0, 0))],
        out_specs=[pl.BlockSpec((N, C), lambda: (0, 0)),
                   pl.BlockSpec((N, C), lambda: (0, 0)),
                   pl.BlockSpec((N, C), lambda: (0, 0))],
        out_shape=[jax.ShapeDtypeStruct((N, C), _f32),
                   jax.ShapeDtypeStruct((N, C), _bf16),
                   jax.ShapeDtypeStruct((N, C), _bf16)],
        grid=(),
    )(at_no, embed_pad)


def _layer_pre_body(use_gate, s_ref, v_ref, ws_ref, wg_ref, hs_ref, lo_ref):
    hs = _dot_ref(s_ref[...], ws_ref[...])
    if use_gate:
        vn2 = jnp.zeros((N, C), _f32)
        for k in range(9):
            vk = v_ref[:, k * C:(k + 1) * C]
            vn2 = vn2 + vk * vk
        vn = jnp.sqrt(vn2 + 1e-6)
        hs = hs * _sigmoid(_dot_ref(vn, wg_ref[...]))
    hi, lo = _split(hs)
    hs_ref[...] = hi
    lo_ref[...] = lo


def _layer_pre(s, vflat, w_self_i, w_gate_i, use_gate):
    return pl.pallas_call(
        functools.partial(_layer_pre_body, use_gate),
        in_specs=[pl.BlockSpec((N, C), lambda: (0, 0)),
                  pl.BlockSpec((N, 9 * C), lambda: (0, 0)),
                  pl.BlockSpec((C, C), lambda: (0, 0)),
                  pl.BlockSpec((C, C), lambda: (0, 0))],
        out_specs=[pl.BlockSpec((N, C), lambda: (0, 0)),
                   pl.BlockSpec((N, C), lambda: (0, 0))],
        out_shape=[jax.ShapeDtypeStruct((N, C), _bf16),
                   jax.ShapeDtypeStruct((N, C), _bf16)],
        grid=(),
    )(s, vflat, w_self_i, w_gate_i)


def _edge_mv_body(do_v, src_ref, w_ref, rsh_ref, hs_ref, hslo_ref, mv_ref):
    oht_s = _onehot_T(src_ref[0], N, BE, _bf16)
    hsg = _gather2(oht_s, hs_ref[...], hslo_ref[...])
    m = w_ref[...] * hsg
    if do_v:
        rsh = rsh_ref[...]
        mv_ref[...] = jnp.concatenate(
            [m] + [m * rsh[:, k:k + 1] for k in range(1, 9)], axis=1)
    else:
        mv_ref[...] = m


def _edge_mv(src3, w_all, rsh_e, hs_bf, hs_lo, layer, do_v):
    nblk = E // BE
    width = 9 * C if do_v else C
    return pl.pallas_call(
        functools.partial(_edge_mv_body, do_v),
        grid=(nblk,),
        in_specs=[
            pl.BlockSpec((1, 1, BE), lambda i: (i, 0, 0)),
            pl.BlockSpec((BE, C), lambda i, L=layer: (i, L)),
            pl.BlockSpec((BE, 16), lambda i: (i, 0)),
            pl.BlockSpec((N, C), lambda i: (0, 0)),
            pl.BlockSpec((N, C), lambda i: (0, 0)),
        ],
        out_specs=pl.BlockSpec((BE, width), lambda i: (i, 0)),
        out_shape=jax.ShapeDtypeStruct((E, width), _f32),
    )(src3, w_all, rsh_e, hs_bf, hs_lo)


_SC_K = 128


def _sc_scatter(R, NR):
    info = plsc.get_sparse_core_info()
    NC, NS = info.num_cores, info.num_subcores
    NW = NC * NS
    assert R % (NW * _SC_K) == 0 and NR % NS == 0
    rows_tile = R // NW
    iters = rows_tile // _SC_K
    acc_rows = NR // NS
    mesh = plsc.VectorSubcoreMesh(core_axis_name="c", subcore_axis_name="s")

    assert iters % 2 == 0

    @functools.partial(
        pl.kernel, mesh=mesh,
        out_type=jax.ShapeDtypeStruct((NC, NR, C), jnp.float32),
        scratch_types=[
            pltpu.VMEM((_SC_K,), jnp.int32),
            pltpu.VMEM((_SC_K,), jnp.int32),
            pltpu.VMEM((_SC_K, C), jnp.float32),
            pltpu.VMEM((_SC_K, C), jnp.float32),
            pltpu.SemaphoreType.DMA,
            pltpu.SemaphoreType.DMA,
            pltpu.SemaphoreType.DMA,
            pltpu.SemaphoreType.DMA,
            pltpu.VMEM_SHARED((NR, C), jnp.float32),
        ],
    )
    def k(vals_hbm, idx_hbm, zeros_hbm, out_hbm, ibuf0, ibuf1, vbuf0, vbuf1,
          si0, si1, sv0, sv1, shared):
        cid = lax.axis_index("c")
        sid = lax.axis_index("s")
        wid = sid * NC + cid
        ibufs, vbufs = (ibuf0, ibuf1), (vbuf0, vbuf1)
        sis, svs = (si0, si1), (sv0, sv1)
        row0 = wid * rows_tile

        pltpu.async_copy(idx_hbm.at[pl.ds(row0, _SC_K)], ibuf0, si0)
        pltpu.async_copy(vals_hbm.at[pl.ds(row0, _SC_K)], vbuf0, sv0)
        pltpu.sync_copy(zeros_hbm.at[pl.ds(sid * acc_rows, acc_rows)],
                        shared.at[pl.ds(sid * acc_rows, acc_rows)])
        plsc.subcore_barrier()

        def body(j, carry):
            for b in range(2):
                it = 2 * j + b
                base = row0 + it * _SC_K
                pltpu.make_async_copy(idx_hbm.at[pl.ds(base, _SC_K)],
                                      ibufs[b], sis[b]).wait()
                pltpu.make_async_copy(vals_hbm.at[pl.ds(base, _SC_K)],
                                      vbufs[b], svs[b]).wait()

                @pl.when(it + 1 < iters)
                def _prefetch(b=b, it=it):
                    nbase = row0 + (it + 1) * _SC_K
                    pltpu.async_copy(idx_hbm.at[pl.ds(nbase, _SC_K)],
                                     ibufs[1 - b], sis[1 - b])
                    pltpu.async_copy(vals_hbm.at[pl.ds(nbase, _SC_K)],
                                     vbufs[1 - b], svs[1 - b])

                pltpu.sync_copy(vbufs[b], shared.at[ibufs[b]], add=True)
            return carry

        lax.fori_loop(0, iters // 2, body, 0)
        plsc.subcore_barrier()
        pltpu.sync_copy(shared.at[pl.ds(sid * acc_rows, acc_rows)],
                        out_hbm.at[cid, pl.ds(sid * acc_rows, acc_rows)])

    return k


def _layer_post_body(tail, has_v, s_ref, v_ref, p0_ref, p1_ref, wu1_ref,
                     wu2_ref, wab_ref, wn1_ref, wn2_ref, *out_refs):
    refs = list(out_refs)
    s_out = refs.pop(0)
    if has_v:
        agg_s = p0_ref[:, :C] + p1_ref[:, :C]
        v_out = refs.pop(0)
        v_out[...] = v_ref[...] + (p0_ref[...] + p1_ref[...])
    else:
        agg_s = p0_ref[...] + p1_ref[...]
    up = _silu(_dot_ref(agg_s, wu1_ref[...]))
    s_new = s_ref[...] + _dot_ref(up, wu2_ref[...])
    s_out[...] = s_new
    if tail:
        a12h_ref, a12l_ref, hn_ref = refs
        a12 = _dot_ref(s_new, wab_ref[...])
        hi, lo = _split(a12)
        a12h_ref[...] = hi
        a12l_ref[...] = lo
        h1 = _silu(_dot_ref(s_new, wn1_ref[...]))
        hn_ref[...] = _dot_ref(h1, wn2_ref[...])


def _layer_post(s, vflat, p0, p1, wu1, wu2, wab, wn1, wn2, tail, has_v):
    W = 9 * C if has_v else C
    out_specs = [pl.BlockSpec((N, C), lambda: (0, 0))]
    out_shape = [jax.ShapeDtypeStruct((N, C), _f32)]
    if has_v:
        out_specs.append(pl.BlockSpec((N, 9 * C), lambda: (0, 0)))
        out_shape.append(jax.ShapeDtypeStruct((N, 9 * C), _f32))
    if tail:
        out_specs += [pl.BlockSpec((N, 2 * C), lambda: (0, 0)),
                      pl.BlockSpec((N, 2 * C), lambda: (0, 0)),
                      pl.BlockSpec((N, 9 * S), lambda: (0, 0))]
        out_shape += [jax.ShapeDtypeStruct((N, 2 * C), _bf16),
                      jax.ShapeDtypeStruct((N, 2 * C), _bf16),
                      jax.ShapeDtypeStruct((N, 9 * S), _f32)]
    return pl.pallas_call(
        functools.partial(_layer_post_body, tail, has_v),
        in_specs=[pl.BlockSpec((N, C), lambda: (0, 0)),
                  pl.BlockSpec((N, 9 * C), lambda: (0, 0)),
                  pl.BlockSpec((N, W), lambda: (0, 0)),
                  pl.BlockSpec((N, W), lambda: (0, 0)),
                  pl.BlockSpec((C, C), lambda: (0, 0)),
                  pl.BlockSpec((C, C), lambda: (0, 0)),
                  pl.BlockSpec((C, 2 * C), lambda: (0, 0)),
                  pl.BlockSpec((C, H), lambda: (0, 0)),
                  pl.BlockSpec((H, 9 * S), lambda: (0, 0))],
        out_specs=out_specs,
        out_shape=out_shape,
        grid=(),
    )(s, vflat, p0, p1, wu1, wu2, wab, wn1, wn2)


def _edge_he_body(src_ref, dst_ref, fr2_ref, frsh_ref, pg_ref,
                  ash_ref, adh_ref,
                  we2a_ref, we2b_ref, weo_ref, emat_ref, nacc_ref):
    i = pl.program_id(0)
    oht_s = _onehot_T(src_ref[0], N, BF, _bf16)
    oht_d = _onehot_T(dst_ref[0], N, BF, _bf16)
    gs = _gather(oht_s, ash_ref[...])
    gd = _gather(oht_d, adh_ref[...])
    fr2 = fr2_ref[...]
    g0 = gs[:, :C] + gd[:, :C] + fr2[:, :C]
    g1 = gs[:, C:] + gd[:, C:] + fr2[:, C:]
    he = _dot_ref(_silu(g0), we2a_ref[...]) + _dot_ref(_silu(g1), we2b_ref[...])
    frsh = frsh_ref[...]
    acc = jnp.concatenate(
        [he[:, k * S:(k + 1) * S] * frsh[:, k:k + 1] for k in range(9)], axis=1)
    emat_ref[...] = _dot_ref(acc * pg_ref[...], weo_ref[...])

    @pl.when(i == 0)
    def _():
        nacc_ref[...] = jnp.zeros_like(nacc_ref)

    nacc_ref[...] += _scatter(oht_d, acc.astype(_bf16))


def _edge_he(fsrc3, fdst3, fr2, frsh, pg, ash, adh, we2a, we2b, weo):
    nblk = EF // BF
    return pl.pallas_call(
        _edge_he_body,
        grid=(nblk,),
        in_specs=[
            pl.BlockSpec((1, 1, BF), lambda i: (i, 0, 0)),
            pl.BlockSpec((1, 1, BF), lambda i: (i, 0, 0)),
            pl.BlockSpec((BF, NA * C), lambda i: (i, 0)),
            pl.BlockSpec((BF, 16), lambda i: (i, 0)),
            pl.BlockSpec((BF, 9 * S), lambda i: (i, 0)),
            pl.BlockSpec((N, NA * C), lambda i: (0, 0)),
            pl.BlockSpec((N, NA * C), lambda i: (0, 0)),
            pl.BlockSpec((C, 9 * S), lambda i: (0, 0)),
            pl.BlockSpec((C, 9 * S), lambda i: (0, 0)),
            pl.BlockSpec((9 * S, B * B), lambda i: (0, 0)),
        ],
        out_specs=[
            pl.BlockSpec((BF, B * B), lambda i: (i, 0)),
            pl.BlockSpec((N, 9 * S), lambda i: (0, 0)),
        ],
        out_shape=[
            jax.ShapeDtypeStruct((EF, B * B), _f32),
            jax.ShapeDtypeStruct((N, 9 * S), _f32),
        ],
    )(fsrc3, fdst3, fr2, frsh, pg, ash, adh, we2a, we2b, weo)


def _node_out_body(s0_ref, hn0_ref, hn1_ref, nacc_ref, wg0_ref, wno_ref, out_ref):
    node_sph = hn0_ref[...] + hn1_ref[...] + nacc_ref[...]
    g0 = _silu(_dot_ref(s0_ref[...], wg0_ref[...]))
    out_ref[...] = _dot_ref(node_sph * g0, wno_ref[...])


def _node_out(s0, hn0, hn1, nacc, wg0, wno):
    return pl.pallas_call(
        _node_out_body,
        in_specs=[pl.BlockSpec((N, C), lambda: (0, 0)),
                  pl.BlockSpec((N, 9 * S), lambda: (0, 0)),
                  pl.BlockSpec((N, 9 * S), lambda: (0, 0)),
                  pl.BlockSpec((N, 9 * S), lambda: (0, 0)),
                  pl.BlockSpec((C, 9 * S), lambda: (0, 0)),
                  pl.BlockSpec((9 * S, B * B), lambda: (0, 0))],
        out_specs=pl.BlockSpec((N, B * B), lambda: (0, 0)),
        out_shape=jax.ShapeDtypeStruct((N, B * B), _f32),
        grid=(),
    )(s0, hn0, hn1, nacc, wg0, wno)


def kernel(at_no, pos, edge_index, fc_edge_index, embed_table, W_filt, b_filt,
           W_self, W_gate, W_up1, W_up2, Wn1, Wn2, We1, We2, Wg0, Wnode_out,
           Wp, Wedge_out):
    src3 = edge_index[0].reshape(E // BE, 1, BE).astype(jnp.int32)
    dst3 = edge_index[1].reshape(E // BE, 1, BE).astype(jnp.int32)
    fsrc3 = fc_edge_index[0].reshape(EF // BF, 1, BF).astype(jnp.int32)
    fdst3 = fc_edge_index[1].reshape(EF // BF, 1, BF).astype(jnp.int32)
    pos_pad = jnp.zeros((N, 8), _f32).at[:, :3].set(pos)
    embed_pad = jnp.zeros((128, C), _f32).at[:100].set(embed_table)
    wf_flat = jnp.transpose(W_filt, (1, 0, 2)).reshape(NB, NL * C)
    b2d = b_filt.reshape(1, NL * C)
    wc_cat = jnp.transpose(We1[:, 2 * C:, :], (1, 0, 2)).reshape(NB, NA * C)

    s0, s0_hi, s0_lo = _embed(at_no.astype(jnp.int32), embed_pad)

    w_all, rsh_e = _geom_e(src3, dst3, pos_pad, wf_flat, b2d)
    del s0_lo
    fr2, frsh, pg = _geom_ef(fsrc3, fdst3, pos_pad, s0_hi, wc_cat, Wp)

    dst_i32 = edge_index[1].astype(jnp.int32)
    idx9 = (dst_i32[:, None] * 9
            + jnp.arange(9, dtype=jnp.int32)[None, :]).reshape(E * 9)
    zeros9 = jnp.zeros((9 * N, C), _f32)
    zeros1 = jnp.zeros((N, C), _f32)
    scat9 = _sc_scatter(E * 9, 9 * N)
    scat1 = _sc_scatter(E, N)

    s = s0
    vflat = jnp.zeros((N, 9 * C), _f32)
    a12h, a12l, hn = [], [], []
    for idx in range(NL):
        has_v = idx < NL - 1
        hs_bf, hs_lo = _layer_pre(s, vflat, W_self[idx], W_gate[idx],
                                  use_gate=idx > 0)
        mv = _edge_mv(src3, w_all, rsh_e, hs_bf, hs_lo, idx, do_v=has_v)
        if has_v:
            parts = scat9(mv.reshape(E * 9, C), idx9, zeros9)
            p0 = parts[0].reshape(N, 9 * C)
            p1 = parts[1].reshape(N, 9 * C)
        else:
            parts = scat1(mv, dst_i32, zeros1)
            p0, p1 = parts[0], parts[1]
        tail = idx >= NL - NA
        j = idx - (NL - NA)
        wab = (jnp.concatenate([We1[j, :C, :], We1[j, C:2 * C, :]], axis=1)
               if tail else jnp.zeros((C, 2 * C), _f32))
        outs = _layer_post(
            s, vflat, p0, p1, W_up1[idx], W_up2[idx], wab,
            Wn1[j] if tail else jnp.zeros((C, H), _f32),
            Wn2[j] if tail else jnp.zeros((H, 9 * S), _f32), tail, has_v)
        outs = list(outs)
        s = outs.pop(0)
        if has_v:
            vflat = outs.pop(0)
        if tail:
            a12h_i, a12l_i, hn_i = outs
            a12h.append(a12h_i)
            a12l.append(a12l_i)
            hn.append(hn_i)

    del a12l
    ash = jnp.concatenate([a12h[0][:, :C], a12h[1][:, :C]], axis=1)
    adh = jnp.concatenate([a12h[0][:, C:], a12h[1][:, C:]], axis=1)
    emat, nacc = _edge_he(fsrc3, fdst3, fr2, frsh, pg, ash, adh,
                          We2[0], We2[1], Wedge_out)
    nmat = _node_out(s0, hn[0], hn[1], nacc, Wg0, Wnode_out)
    return nmat.reshape(N, B, B), emat.reshape(EF, B, B)


# speedup vs baseline: 1.4122x; 1.2881x over previous
import functools
import jax
import jax.numpy as jnp
from jax import lax
from jax.experimental import pallas as pl
from jax.experimental.pallas import tpu as pltpu, tpu_sc as plsc

N = 1024
E = 16384
EF = 65536
C = 128
NB = 32
H = 128
S = 32
B = 16
NL = 4
NA = 2
CUTOFF = 5.0

BE = 2048
BF = 2048

_bf16 = jnp.bfloat16
_f32 = jnp.float32
_HI = lax.Precision.HIGHEST


def _sigmoid(x):
    return 1.0 / (1.0 + jnp.exp(-x))


def _silu(x):
    return x * _sigmoid(x)


def _dot_ref(a, b):
    return jnp.dot(a.astype(_bf16), b.astype(_bf16), preferred_element_type=_f32)


def _onehot_T(idx2d, rows, cols, dtype):
    return (lax.broadcasted_iota(jnp.int32, (rows, cols), 0) == idx2d).astype(dtype)


def _gather(ohT, table, precision=None):
    return lax.dot_general(ohT, table, (((0,), (0,)), ((), ())),
                           preferred_element_type=_f32, precision=precision)


def _gather2(ohT, hi, lo):
    return _gather(ohT, hi) + _gather(ohT, lo)


def _scatter(ohT, vals):
    return jnp.dot(ohT, vals, preferred_element_type=_f32)


def _split(x):
    hi = x.astype(_bf16)
    return hi, (x - hi.astype(_f32)).astype(_bf16)


def _edge_geom(oht_s, oht_d, pos_tri):
    parts = _gather(oht_d - oht_s, pos_tri)
    vec = parts[:, 0:8] + parts[:, 8:16] + parts[:, 16:24]
    x, y, z = vec[:, 0:1], vec[:, 1:2], vec[:, 2:3]
    d = jnp.sqrt(x * x + y * y + z * z + 1e-12)
    return vec, d


def _rbf_block(d):
    n = lax.broadcasted_iota(jnp.int32, (1, NB), 1).astype(_f32) + 1.0
    xc = d / CUTOFF
    rbf = jnp.sqrt(2.0 / CUTOFF) * jnp.sin(n * (jnp.pi * xc)) / d
    u = jnp.clip(xc, 0.0, 1.0)
    fc = 1.0 - 10.0 * u ** 3 + 15.0 * u ** 4 - 6.0 * u ** 5
    return rbf * fc


def _rsh16(vec, d):
    u = vec / d
    x, y, z = u[:, 0:1], u[:, 1:2], u[:, 2:3]
    s3 = jnp.sqrt(3.0)
    cols = [jnp.ones_like(x), x, y, z, s3 * x * y, s3 * y * z,
            0.5 * (3.0 * z * z - 1.0), s3 * x * z, 0.5 * s3 * (x * x - y * y)]
    blk = x.shape[0]
    out = jnp.zeros((blk, 16), _f32)
    for k, c in enumerate(cols):
        sel = (lax.broadcasted_iota(jnp.int32, (1, 16), 1) == k).astype(_f32)
        out = out + c * sel
    return out


def _geom_e_body(src_ref, dst_ref, pos_ref, wf_ref, bf_ref, w_ref, rsh_ref):
    oht_s = _onehot_T(src_ref[0], N, BE, _bf16)
    oht_d = _onehot_T(dst_ref[0], N, BE, _bf16)
    vec, d = _edge_geom(oht_s, oht_d, pos_ref[...])
    rbf = _rbf_block(d)
    w_ref[...] = _silu(_dot_ref(rbf, wf_ref[...]) + bf_ref[...])
    rsh_ref[...] = _rsh16(vec, d)


def _geom_e(src3, dst3, pos_pad, wf_flat, b2d):
    nblk = E // BE
    return pl.pallas_call(
        _geom_e_body,
        grid=(nblk,),
        in_specs=[
            pl.BlockSpec((1, 1, BE), lambda i: (i, 0, 0)),
            pl.BlockSpec((1, 1, BE), lambda i: (i, 0, 0)),
            pl.BlockSpec((N, 24), lambda i: (0, 0)),
            pl.BlockSpec((NB, NL * C), lambda i: (0, 0)),
            pl.BlockSpec((1, NL * C), lambda i: (0, 0)),
        ],
        out_specs=[
            pl.BlockSpec((BE, NL * C), lambda i: (i, 0)),
            pl.BlockSpec((BE, 16), lambda i: (i, 0)),
        ],
        out_shape=[
            jax.ShapeDtypeStruct((E, NL * C), _f32),
            jax.ShapeDtypeStruct((E, 16), _f32),
        ],
    )(src3, dst3, pos_pad, wf_flat, b2d)


def _geom_ef_body(src_ref, dst_ref, pos_ref, s0h_ref, wc_ref, wp_ref,
                  fr2_ref, frsh_ref, pg_ref):
    oht_s = _onehot_T(src_ref[0], N, BF, _bf16)
    oht_d = _onehot_T(dst_ref[0], N, BF, _bf16)
    vec, d = _edge_geom(oht_s, oht_d, pos_ref[...])
    rbf = _rbf_block(d)
    fr2_ref[...] = _dot_ref(rbf, wc_ref[...])
    frsh_ref[...] = _rsh16(vec, d)
    n0sum = _gather(oht_s + oht_d, s0h_ref[...])
    pg_ref[...] = _silu(_dot_ref(n0sum, wp_ref[...]))


def _geom_ef(fsrc3, fdst3, pos_pad, s0_hi, wc_cat, wp):
    nblk = EF // BF
    return pl.pallas_call(
        _geom_ef_body,
        grid=(nblk,),
        in_specs=[
            pl.BlockSpec((1, 1, BF), lambda i: (i, 0, 0)),
            pl.BlockSpec((1, 1, BF), lambda i: (i, 0, 0)),
            pl.BlockSpec((N, 24), lambda i: (0, 0)),
            pl.BlockSpec((N, C), lambda i: (0, 0)),
            pl.BlockSpec((NB, NA * C), lambda i: (0, 0)),
            pl.BlockSpec((C, 9 * S), lambda i: (0, 0)),
        ],
        out_specs=[
            pl.BlockSpec((BF, NA * C), lambda i: (i, 0)),
            pl.BlockSpec((BF, 16), lambda i: (i, 0)),
            pl.BlockSpec((BF, 9 * S), lambda i: (i, 0)),
        ],
        out_shape=[
            jax.ShapeDtypeStruct((EF, NA * C), _f32),
            jax.ShapeDtypeStruct((EF, 16), _f32),
            jax.ShapeDtypeStruct((EF, 9 * S), _f32),
        ],
    )(fsrc3, fdst3, pos_pad, s0_hi, wc_cat, wp)


def _embed_body(at_ref, tab_ref, pos_ref, s0_ref, hi_ref, lo_ref, tri_ref):
    idx2d = jnp.reshape(at_ref[...], (1, N))
    oht = _onehot_T(idx2d, 128, N, _f32)
    s0 = _gather(oht, tab_ref[...], _HI)
    s0_ref[...] = s0
    hi, lo = _split(s0)
    hi_ref[...] = hi
    lo_ref[...] = lo
    p = pos_ref[...]
    phi = p.astype(_bf16)
    r = p - phi.astype(_f32)
    plo = r.astype(_bf16)
    plo2 = (r - plo.astype(_f32)).astype(_bf16)
    tri_ref[...] = jnp.concatenate([phi, plo, plo2], axis=1)


def _embed(at_no, embed_pad, pos_pad):
    return pl.pallas_call(
        _embed_body,
        in_specs=[pl.BlockSpec((N,), lambda: (0,)),
                  pl.BlockSpec((128, C), lambda: (0, 0)),
                  pl.BlockSpec((N, 8), lambda: (0, 0))],
        out_specs=[pl.BlockSpec((N, C), lambda: (0, 0)),
                   pl.BlockSpec((N, C), lambda: (0, 0)),
                   pl.BlockSpec((N, C), lambda: (0, 0)),
                   pl.BlockSpec((N, 24), lambda: (0, 0))],
        out_shape=[jax.ShapeDtypeStruct((N, C), _f32),
                   jax.ShapeDtypeStruct((N, C), _bf16),
                   jax.ShapeDtypeStruct((N, C), _bf16),
                   jax.ShapeDtypeStruct((N, 24), _bf16)],
        grid=(),
    )(at_no, embed_pad, pos_pad)


def _layer_pre_body(use_gate, s_ref, v_ref, ws_ref, wg_ref, hs_ref, lo_ref):
    hs = _dot_ref(s_ref[...], ws_ref[...])
    if use_gate:
        vn2 = jnp.zeros((N, C), _f32)
        for k in range(9):
            vk = v_ref[:, k * C:(k + 1) * C]
            vn2 = vn2 + vk * vk
        vn = jnp.sqrt(vn2 + 1e-6)
        hs = hs * _sigmoid(_dot_ref(vn, wg_ref[...]))
    hi, lo = _split(hs)
    hs_ref[...] = hi
    lo_ref[...] = lo


def _layer_pre(s, vflat, w_self_i, w_gate_i, use_gate):
    return pl.pallas_call(
        functools.partial(_layer_pre_body, use_gate),
        in_specs=[pl.BlockSpec((N, C), lambda: (0, 0)),
                  pl.BlockSpec((N, 9 * C), lambda: (0, 0)),
                  pl.BlockSpec((C, C), lambda: (0, 0)),
                  pl.BlockSpec((C, C), lambda: (0, 0))],
        out_specs=[pl.BlockSpec((N, C), lambda: (0, 0)),
                   pl.BlockSpec((N, C), lambda: (0, 0))],
        out_shape=[jax.ShapeDtypeStruct((N, C), _bf16),
                   jax.ShapeDtypeStruct((N, C), _bf16)],
        grid=(),
    )(s, vflat, w_self_i, w_gate_i)


def _edge_mv_body(do_v, src_ref, w_ref, rsh_ref, hs_ref, hslo_ref, mv_ref):
    oht_s = _onehot_T(src_ref[0], N, BE, _bf16)
    hsg = _gather2(oht_s, hs_ref[...], hslo_ref[...])
    m = w_ref[...] * hsg
    if do_v:
        rsh = rsh_ref[...]
        mv_ref[...] = jnp.concatenate(
            [m] + [m * rsh[:, k:k + 1] for k in range(1, 9)], axis=1)
    else:
        mv_ref[...] = m


def _edge_mv(src3, w_all, rsh_e, hs_bf, hs_lo, layer, do_v):
    nblk = E // BE
    width = 9 * C if do_v else C
    return pl.pallas_call(
        functools.partial(_edge_mv_body, do_v),
        grid=(nblk,),
        in_specs=[
            pl.BlockSpec((1, 1, BE), lambda i: (i, 0, 0)),
            pl.BlockSpec((BE, C), lambda i, L=layer: (i, L)),
            pl.BlockSpec((BE, 16), lambda i: (i, 0)),
            pl.BlockSpec((N, C), lambda i: (0, 0)),
            pl.BlockSpec((N, C), lambda i: (0, 0)),
        ],
        out_specs=pl.BlockSpec((BE, width), lambda i: (i, 0)),
        out_shape=jax.ShapeDtypeStruct((E, width), _f32),
    )(src3, w_all, rsh_e, hs_bf, hs_lo)


_SC_K = 128


def _sc_scatter(R, NR):
    info = plsc.get_sparse_core_info()
    NC, NS = info.num_cores, info.num_subcores
    NW = NC * NS
    assert R % (NW * _SC_K) == 0 and NR % NS == 0
    rows_tile = R // NW
    iters = rows_tile // _SC_K
    acc_rows = NR // NS
    mesh = plsc.VectorSubcoreMesh(core_axis_name="c", subcore_axis_name="s")

    assert iters % 2 == 0

    @functools.partial(
        pl.kernel, mesh=mesh,
        out_type=jax.ShapeDtypeStruct((NC, NR, C), jnp.float32),
        scratch_types=[
            pltpu.VMEM((_SC_K,), jnp.int32),
            pltpu.VMEM((_SC_K,), jnp.int32),
            pltpu.VMEM((_SC_K, C), jnp.float32),
            pltpu.VMEM((_SC_K, C), jnp.float32),
            pltpu.SemaphoreType.DMA,
            pltpu.SemaphoreType.DMA,
            pltpu.SemaphoreType.DMA,
            pltpu.SemaphoreType.DMA,
            pltpu.VMEM_SHARED((NR, C), jnp.float32),
        ],
    )
    def k(vals_hbm, idx_hbm, zeros_hbm, out_hbm, ibuf0, ibuf1, vbuf0, vbuf1,
          si0, si1, sv0, sv1, shared):
        cid = lax.axis_index("c")
        sid = lax.axis_index("s")
        wid = sid * NC + cid
        ibufs, vbufs = (ibuf0, ibuf1), (vbuf0, vbuf1)
        sis, svs = (si0, si1), (sv0, sv1)
        row0 = wid * rows_tile

        pltpu.async_copy(idx_hbm.at[pl.ds(row0, _SC_K)], ibuf0, si0)
        pltpu.async_copy(vals_hbm.at[pl.ds(row0, _SC_K)], vbuf0, sv0)
        pltpu.sync_copy(zeros_hbm.at[pl.ds(sid * acc_rows, acc_rows)],
                        shared.at[pl.ds(sid * acc_rows, acc_rows)])
        plsc.subcore_barrier()

        def body(j, carry):
            for b in range(2):
                it = 2 * j + b
                base = row0 + it * _SC_K
                pltpu.make_async_copy(idx_hbm.at[pl.ds(base, _SC_K)],
                                      ibufs[b], sis[b]).wait()
                pltpu.make_async_copy(vals_hbm.at[pl.ds(base, _SC_K)],
                                      vbufs[b], svs[b]).wait()

                @pl.when(it + 1 < iters)
                def _prefetch(b=b, it=it):
                    nbase = row0 + (it + 1) * _SC_K
                    pltpu.async_copy(idx_hbm.at[pl.ds(nbase, _SC_K)],
                                     ibufs[1 - b], sis[1 - b])
                    pltpu.async_copy(vals_hbm.at[pl.ds(nbase, _SC_K)],
                                     vbufs[1 - b], svs[1 - b])

                pltpu.sync_copy(vbufs[b], shared.at[ibufs[b]], add=True)
            return carry

        lax.fori_loop(0, iters // 2, body, 0)
        plsc.subcore_barrier()
        pltpu.sync_copy(shared.at[pl.ds(sid * acc_rows, acc_rows)],
                        out_hbm.at[cid, pl.ds(sid * acc_rows, acc_rows)])

    return k


def _layer_post_body(tail, has_v, s_ref, v_ref, p0_ref, p1_ref, wu1_ref,
                     wu2_ref, wab_ref, wn1_ref, wn2_ref, *out_refs):
    refs = list(out_refs)
    s_out = refs.pop(0)
    if has_v:
        agg_s = p0_ref[:, :C] + p1_ref[:, :C]
        v_out = refs.pop(0)
        v_out[...] = v_ref[...] + (p0_ref[...] + p1_ref[...])
    else:
        agg_s = p0_ref[...] + p1_ref[...]
    up = _silu(_dot_ref(agg_s, wu1_ref[...]))
    s_new = s_ref[...] + _dot_ref(up, wu2_ref[...])
    s_out[...] = s_new
    if tail:
        a12h_ref, a12l_ref, hn_ref = refs
        a12 = _dot_ref(s_new, wab_ref[...])
        hi, lo = _split(a12)
        a12h_ref[...] = hi
        a12l_ref[...] = lo
        h1 = _silu(_dot_ref(s_new, wn1_ref[...]))
        hn_ref[...] = _dot_ref(h1, wn2_ref[...])


def _layer_post(s, vflat, p0, p1, wu1, wu2, wab, wn1, wn2, tail, has_v):
    W = 9 * C if has_v else C
    out_specs = [pl.BlockSpec((N, C), lambda: (0, 0))]
    out_shape = [jax.ShapeDtypeStruct((N, C), _f32)]
    if has_v:
        out_specs.append(pl.BlockSpec((N, 9 * C), lambda: (0, 0)))
        out_shape.append(jax.ShapeDtypeStruct((N, 9 * C), _f32))
    if tail:
        out_specs += [pl.BlockSpec((N, 2 * C), lambda: (0, 0)),
                      pl.BlockSpec((N, 2 * C), lambda: (0, 0)),
                      pl.BlockSpec((N, 9 * S), lambda: (0, 0))]
        out_shape += [jax.ShapeDtypeStruct((N, 2 * C), _bf16),
                      jax.ShapeDtypeStruct((N, 2 * C), _bf16),
                      jax.ShapeDtypeStruct((N, 9 * S), _f32)]
    return pl.pallas_call(
        functools.partial(_layer_post_body, tail, has_v),
        in_specs=[pl.BlockSpec((N, C), lambda: (0, 0)),
                  pl.BlockSpec((N, 9 * C), lambda: (0, 0)),
                  pl.BlockSpec((N, W), lambda: (0, 0)),
                  pl.BlockSpec((N, W), lambda: (0, 0)),
                  pl.BlockSpec((C, C), lambda: (0, 0)),
                  pl.BlockSpec((C, C), lambda: (0, 0)),
                  pl.BlockSpec((C, 2 * C), lambda: (0, 0)),
                  pl.BlockSpec((C, H), lambda: (0, 0)),
                  pl.BlockSpec((H, 9 * S), lambda: (0, 0))],
        out_specs=out_specs,
        out_shape=out_shape,
        grid=(),
    )(s, vflat, p0, p1, wu1, wu2, wab, wn1, wn2)


def _edge_he_body(src_ref, dst_ref, fr2_ref, frsh_ref, pg_ref,
                  ash_ref, adh_ref,
                  we2a_ref, we2b_ref, weo_ref, emat_ref, nacc_ref):
    i = pl.program_id(0)
    oht_s = _onehot_T(src_ref[0], N, BF, _bf16)
    oht_d = _onehot_T(dst_ref[0], N, BF, _bf16)
    gs = _gather(oht_s, ash_ref[...])
    gd = _gather(oht_d, adh_ref[...])
    fr2 = fr2_ref[...]
    g0 = gs[:, :C] + gd[:, :C] + fr2[:, :C]
    g1 = gs[:, C:] + gd[:, C:] + fr2[:, C:]
    he = _dot_ref(_silu(g0), we2a_ref[...]) + _dot_ref(_silu(g1), we2b_ref[...])
    frsh = frsh_ref[...]
    acc = jnp.concatenate(
        [he[:, k * S:(k + 1) * S] * frsh[:, k:k + 1] for k in range(9)], axis=1)
    emat_ref[...] = _dot_ref(acc * pg_ref[...], weo_ref[...])

    @pl.when(i == 0)
    def _():
        nacc_ref[...] = jnp.zeros_like(nacc_ref)

    nacc_ref[...] += _scatter(oht_d, acc.astype(_bf16))


def _edge_he(fsrc3, fdst3, fr2, frsh, pg, ash, adh, we2a, we2b, weo):
    nblk = EF // BF
    return pl.pallas_call(
        _edge_he_body,
        grid=(nblk,),
        in_specs=[
            pl.BlockSpec((1, 1, BF), lambda i: (i, 0, 0)),
            pl.BlockSpec((1, 1, BF), lambda i: (i, 0, 0)),
            pl.BlockSpec((BF, NA * C), lambda i: (i, 0)),
            pl.BlockSpec((BF, 16), lambda i: (i, 0)),
            pl.BlockSpec((BF, 9 * S), lambda i: (i, 0)),
            pl.BlockSpec((N, NA * C), lambda i: (0, 0)),
            pl.BlockSpec((N, NA * C), lambda i: (0, 0)),
            pl.BlockSpec((C, 9 * S), lambda i: (0, 0)),
            pl.BlockSpec((C, 9 * S), lambda i: (0, 0)),
            pl.BlockSpec((9 * S, B * B), lambda i: (0, 0)),
        ],
        out_specs=[
            pl.BlockSpec((BF, B * B), lambda i: (i, 0)),
            pl.BlockSpec((N, 9 * S), lambda i: (0, 0)),
        ],
        out_shape=[
            jax.ShapeDtypeStruct((EF, B * B), _f32),
            jax.ShapeDtypeStruct((N, 9 * S), _f32),
        ],
    )(fsrc3, fdst3, fr2, frsh, pg, ash, adh, we2a, we2b, weo)


def _node_out_body(s0_ref, hn0_ref, hn1_ref, nacc_ref, wg0_ref, wno_ref, out_ref):
    node_sph = hn0_ref[...] + hn1_ref[...] + nacc_ref[...]
    g0 = _silu(_dot_ref(s0_ref[...], wg0_ref[...]))
    out_ref[...] = _dot_ref(node_sph * g0, wno_ref[...])


def _node_out(s0, hn0, hn1, nacc, wg0, wno):
    return pl.pallas_call(
        _node_out_body,
        in_specs=[pl.BlockSpec((N, C), lambda: (0, 0)),
                  pl.BlockSpec((N, 9 * S), lambda: (0, 0)),
                  pl.BlockSpec((N, 9 * S), lambda: (0, 0)),
                  pl.BlockSpec((N, 9 * S), lambda: (0, 0)),
                  pl.BlockSpec((C, 9 * S), lambda: (0, 0)),
                  pl.BlockSpec((9 * S, B * B), lambda: (0, 0))],
        out_specs=pl.BlockSpec((N, B * B), lambda: (0, 0)),
        out_shape=jax.ShapeDtypeStruct((N, B * B), _f32),
        grid=(),
    )(s0, hn0, hn1, nacc, wg0, wno)


def kernel(at_no, pos, edge_index, fc_edge_index, embed_table, W_filt, b_filt,
           W_self, W_gate, W_up1, W_up2, Wn1, Wn2, We1, We2, Wg0, Wnode_out,
           Wp, Wedge_out):
    src3 = edge_index[0].reshape(E // BE, 1, BE).astype(jnp.int32)
    dst3 = edge_index[1].reshape(E // BE, 1, BE).astype(jnp.int32)
    fsrc3 = fc_edge_index[0].reshape(EF // BF, 1, BF).astype(jnp.int32)
    fdst3 = fc_edge_index[1].reshape(EF // BF, 1, BF).astype(jnp.int32)
    pos_pad = jnp.zeros((N, 8), _f32).at[:, :3].set(pos)
    embed_pad = jnp.zeros((128, C), _f32).at[:100].set(embed_table)
    wf_flat = jnp.transpose(W_filt, (1, 0, 2)).reshape(NB, NL * C)
    b2d = b_filt.reshape(1, NL * C)
    wc_cat = jnp.transpose(We1[:, 2 * C:, :], (1, 0, 2)).reshape(NB, NA * C)

    s0, s0_hi, s0_lo, pos_tri = _embed(at_no.astype(jnp.int32), embed_pad,
                                       pos_pad)

    w_all, rsh_e = _geom_e(src3, dst3, pos_tri, wf_flat, b2d)
    del s0_lo
    fr2, frsh, pg = _geom_ef(fsrc3, fdst3, pos_tri, s0_hi, wc_cat, Wp)

    dst_i32 = edge_index[1].astype(jnp.int32)
    idx9 = (dst_i32[:, None] * 9
            + jnp.arange(9, dtype=jnp.int32)[None, :]).reshape(E * 9)
    zeros9 = jnp.zeros((9 * N, C), _f32)
    zeros1 = jnp.zeros((N, C), _f32)
    scat9 = _sc_scatter(E * 9, 9 * N)
    scat1 = _sc_scatter(E, N)

    s = s0
    vflat = jnp.zeros((N, 9 * C), _f32)
    a12h, a12l, hn = [], [], []
    for idx in range(NL):
        has_v = idx < NL - 1
        hs_bf, hs_lo = _layer_pre(s, vflat, W_self[idx], W_gate[idx],
                                  use_gate=idx > 0)
        mv = _edge_mv(src3, w_all, rsh_e, hs_bf, hs_lo, idx, do_v=has_v)
        if has_v:
            parts = scat9(mv.reshape(E * 9, C), idx9, zeros9)
            p0 = parts[0].reshape(N, 9 * C)
            p1 = parts[1].reshape(N, 9 * C)
        else:
            parts = scat1(mv, dst_i32, zeros1)
            p0, p1 = parts[0], parts[1]
        tail = idx >= NL - NA
        j = idx - (NL - NA)
        wab = (jnp.concatenate([We1[j, :C, :], We1[j, C:2 * C, :]], axis=1)
               if tail else jnp.zeros((C, 2 * C), _f32))
        outs = _layer_post(
            s, vflat, p0, p1, W_up1[idx], W_up2[idx], wab,
            Wn1[j] if tail else jnp.zeros((C, H), _f32),
            Wn2[j] if tail else jnp.zeros((H, 9 * S), _f32), tail, has_v)
        outs = list(outs)
        s = outs.pop(0)
        if has_v:
            vflat = outs.pop(0)
        if tail:
            a12h_i, a12l_i, hn_i = outs
            a12h.append(a12h_i)
            a12l.append(a12l_i)
            hn.append(hn_i)

    del a12l
    ash = jnp.concatenate([a12h[0][:, :C], a12h[1][:, :C]], axis=1)
    adh = jnp.concatenate([a12h[0][:, C:], a12h[1][:, C:]], axis=1)
    emat, nacc = _edge_he(fsrc3, fdst3, fr2, frsh, pg, ash, adh,
                          We2[0], We2[1], Wedge_out)
    nmat = _node_out(s0, hn[0], hn[1], nacc, Wg0, Wnode_out)
    return nmat.reshape(N, B, B), emat.reshape(EF, B, B)
